# baseline (speedup 1.0000x reference)
import jax
import jax.numpy as jnp
from jax import lax
from jax.experimental import pallas as pl
from jax.experimental.pallas import tpu as pltpu
from jax.experimental.pallas import tpu_sc as plsc

NUM_CORES = 2
NUM_SUBCORES = 16
NUM_WORKERS = NUM_CORES * NUM_SUBCORES
LANES = 16

BATCH = 16384
DIM = 128
CHUNK = 128
B_PER_W = BATCH // NUM_WORKERS
N_CHUNKS = B_PER_W // CHUNK


def _dist_mult_body(data_hbm, ent_hbm, rel_hbm, out_hbm,
                    data_v, idx_h, idx_r, idx_t,
                    rows_h, rows_r, rows_t, out_v, sems):
    wid = lax.axis_index("s") * NUM_CORES + lax.axis_index("c")
    base_w = wid * B_PER_W

    lane_ids = lax.iota(jnp.int32, LANES)
    shuffle_idx = [lane_ids ^ s for s in (8, 4, 2, 1)]
    dnums = lax.GatherDimensionNumbers(
        offset_dims=(), collapsed_slice_dims=(0,), start_index_map=(0,))

    def lane_sum(v):
        for idx in shuffle_idx:
            v = v + lax.gather(
                v, idx[:, None], dnums, slice_sizes=(1,),
                mode=lax.GatherScatterMode.PROMISE_IN_BOUNDS)
        return v

    pltpu.sync_copy(data_hbm.at[pl.ds(base_w * 3, B_PER_W * 3)], data_v)

    def shuf(v, idx):
        return lax.gather(v, idx[:, None], dnums, slice_sizes=(1,),
                          mode=lax.GatherScatterMode.PROMISE_IN_BOUNDS)

    deint_pos = [3 * lane_ids + f for f in range(3)]
    deint_lane = [p % LANES for p in deint_pos]
    idx_refs = (idx_h, idx_r, idx_t)

    def deint_body(g, _):
        w0 = data_v[pl.ds(g * 3 * LANES, LANES)]
        w1 = data_v[pl.ds(g * 3 * LANES + LANES, LANES)]
        w2 = data_v[pl.ds(g * 3 * LANES + 2 * LANES, LANES)]
        for f in range(3):
            p, q = deint_pos[f], deint_lane[f]
            v = jnp.where(p < LANES, shuf(w0, q),
                          jnp.where(p < 2 * LANES, shuf(w1, q), shuf(w2, q)))
            idx_refs[f][pl.ds(g * LANES, LANES)] = v
        return 0

    lax.fori_loop(0, B_PER_W // LANES, deint_body, 0)

    def start_chunk(c, buf):
        off = c * CHUNK
        pltpu.make_async_copy(
            ent_hbm.at[idx_h.at[pl.ds(off, CHUNK)]], rows_h.at[buf],
            sems.at[buf]).start()
        pltpu.make_async_copy(
            rel_hbm.at[idx_r.at[pl.ds(off, CHUNK)]], rows_r.at[buf],
            sems.at[buf]).start()
        pltpu.make_async_copy(
            ent_hbm.at[idx_t.at[pl.ds(off, CHUNK)]], rows_t.at[buf],
            sems.at[buf]).start()

    def wait_chunk(c, buf):
        off = c * CHUNK
        pltpu.make_async_copy(
            ent_hbm.at[idx_h.at[pl.ds(off, CHUNK)]], rows_h.at[buf],
            sems.at[buf]).wait()
        pltpu.make_async_copy(
            rel_hbm.at[idx_r.at[pl.ds(off, CHUNK)]], rows_r.at[buf],
            sems.at[buf]).wait()
        pltpu.make_async_copy(
            ent_hbm.at[idx_t.at[pl.ds(off, CHUNK)]], rows_t.at[buf],
            sems.at[buf]).wait()

    def compute_chunk(c, buf):
        base = base_w + c * CHUNK
        rh = rows_h.at[buf]
        rr = rows_r.at[buf]
        rt = rows_t.at[buf]

        def group_body(g, _):
            def triple_body(j, res):
                i = g * LANES + j
                acc = jnp.zeros((LANES,), jnp.float32)
                for k in range(DIM // LANES):
                    hv = rh[i, pl.ds(k * LANES, LANES)]
                    rv = rr[i, pl.ds(k * LANES, LANES)]
                    tv = rt[i, pl.ds(k * LANES, LANES)]
                    acc = acc + hv * rv * tv
                return jnp.where(lane_ids == j, lane_sum(acc), res)

            res = lax.fori_loop(0, LANES, triple_body,
                                jnp.zeros((LANES,), jnp.float32))
            out_v[pl.ds(g * LANES, LANES)] = jnp.clip(res, -20.0, 20.0)
            return 0

        lax.fori_loop(0, CHUNK // LANES, group_body, 0)
        pltpu.sync_copy(out_v, out_hbm.at[pl.ds(base, CHUNK)])

    start_chunk(0, 0)
    for c in range(N_CHUNKS):
        buf = c % 2
        if c + 1 < N_CHUNKS:
            start_chunk(c + 1, 1 - buf)
        wait_chunk(c, buf)
        compute_chunk(c, buf)


@jax.jit
def _dist_mult(data, ent_embs, rel_embs):
    mesh = plsc.VectorSubcoreMesh(core_axis_name="c", subcore_axis_name="s")
    run = pl.kernel(
        _dist_mult_body,
        out_type=jax.ShapeDtypeStruct((BATCH,), jnp.float32),
        mesh=mesh,
        scratch_types=[
            pltpu.VMEM((B_PER_W * 3,), jnp.int32),
            pltpu.VMEM((B_PER_W,), jnp.int32),
            pltpu.VMEM((B_PER_W,), jnp.int32),
            pltpu.VMEM((B_PER_W,), jnp.int32),
            pltpu.VMEM((2, CHUNK, DIM), jnp.float32),
            pltpu.VMEM((2, CHUNK, DIM), jnp.float32),
            pltpu.VMEM((2, CHUNK, DIM), jnp.float32),
            pltpu.VMEM((CHUNK,), jnp.float32),
            pltpu.SemaphoreType.DMA((2,)),
        ],
    )
    return run(data, ent_embs, rel_embs)


def kernel(data, ent_embs, rel_embs):
    flat = data.astype(jnp.int32).reshape(-1)
    return _dist_mult(flat, ent_embs, rel_embs)


# speedup vs baseline: 1.2913x; 1.2913x over previous
import jax
import jax.numpy as jnp
from jax import lax
from jax.experimental import pallas as pl
from jax.experimental.pallas import tpu as pltpu
from jax.experimental.pallas import tpu_sc as plsc

NUM_CORES = 2
NUM_SUBCORES = 16
NUM_WORKERS = NUM_CORES * NUM_SUBCORES
LANES = 16

BATCH = 16384
DIM = 128
CHUNK = 64
B_PER_W = BATCH // NUM_WORKERS
N_CHUNKS = B_PER_W // CHUNK


def _dist_mult_body(ent_hbm, rel_hbm, h_hbm, r_hbm, t_hbm, out_hbm,
                    idx_h, idx_r, idx_t, rows_h, rows_r, rows_t, out_v,
                    sems, idx_sem, out_sem):
    wid = lax.axis_index("s") * NUM_CORES + lax.axis_index("c")
    base_w = wid * B_PER_W

    lane_ids = lax.iota(jnp.int32, LANES)
    shuffle_idx = [lane_ids ^ s for s in (8, 4, 2, 1)]
    dnums = lax.GatherDimensionNumbers(
        offset_dims=(), collapsed_slice_dims=(0,), start_index_map=(0,))

    def lane_sum(v):
        for idx in shuffle_idx:
            v = v + lax.gather(
                v, idx[:, None], dnums, slice_sizes=(1,),
                mode=lax.GatherScatterMode.PROMISE_IN_BOUNDS)
        return v

    cps = [pltpu.make_async_copy(src.at[pl.ds(base_w, B_PER_W)], dst, idx_sem)
           for src, dst in ((h_hbm, idx_h), (r_hbm, idx_r), (t_hbm, idx_t))]
    for cp in cps:
        cp.start()
    for cp in cps:
        cp.wait()

    def chunk_copies(c, buf):
        off = c * CHUNK
        return [
            pltpu.make_async_copy(
                ent_hbm.at[idx_h.at[pl.ds(off, CHUNK)]], rows_h.at[buf],
                sems.at[buf]),
            pltpu.make_async_copy(
                rel_hbm.at[idx_r.at[pl.ds(off, CHUNK)]], rows_r.at[buf],
                sems.at[buf]),
            pltpu.make_async_copy(
                ent_hbm.at[idx_t.at[pl.ds(off, CHUNK)]], rows_t.at[buf],
                sems.at[buf]),
        ]

    def compute_chunk(c, buf):
        base = base_w + c * CHUNK
        rh = rows_h.at[buf]
        rr = rows_r.at[buf]
        rt = rows_t.at[buf]
        ov = out_v.at[buf]

        def triple_sum(i):
            acc = jnp.zeros((LANES,), jnp.float32)
            for k in range(DIM // LANES):
                hv = rh[i, pl.ds(k * LANES, LANES)]
                rv = rr[i, pl.ds(k * LANES, LANES)]
                tv = rt[i, pl.ds(k * LANES, LANES)]
                acc = acc + hv * rv * tv
            return lane_sum(acc)

        def group_body(g, _):
            def pair_body(j, res):
                res = jnp.where(lane_ids == 2 * j,
                                triple_sum(g * LANES + 2 * j), res)
                return jnp.where(lane_ids == 2 * j + 1,
                                 triple_sum(g * LANES + 2 * j + 1), res)

            res = lax.fori_loop(0, LANES // 2, pair_body,
                                jnp.zeros((LANES,), jnp.float32))
            ov[pl.ds(g * LANES, LANES)] = jnp.clip(res, -20.0, 20.0)
            return 0

        lax.fori_loop(0, CHUNK // LANES, group_body, 0)
        pltpu.make_async_copy(ov, out_hbm.at[pl.ds(base, CHUNK)],
                              out_sem).start()

    for cp in chunk_copies(0, 0):
        cp.start()
    for c in range(N_CHUNKS):
        buf = c % 2
        if c + 1 < N_CHUNKS:
            for cp in chunk_copies(c + 1, 1 - buf):
                cp.start()
        for cp in chunk_copies(c, buf):
            cp.wait()
        if c >= 2:
            pltpu.make_async_copy(
                out_v.at[buf],
                out_hbm.at[pl.ds(base_w + (c - 2) * CHUNK, CHUNK)],
                out_sem).wait()
        compute_chunk(c, buf)

    for c in (N_CHUNKS - 2, N_CHUNKS - 1):
        pltpu.make_async_copy(
            out_v.at[c % 2], out_hbm.at[pl.ds(base_w + c * CHUNK, CHUNK)],
            out_sem).wait()


@jax.jit
def _dist_mult(ent_embs, rel_embs, h_idx, r_idx, t_idx):
    mesh = plsc.VectorSubcoreMesh(core_axis_name="c", subcore_axis_name="s")
    run = pl.kernel(
        _dist_mult_body,
        out_type=jax.ShapeDtypeStruct((BATCH,), jnp.float32),
        mesh=mesh,
        scratch_types=[
            pltpu.VMEM((B_PER_W,), jnp.int32),
            pltpu.VMEM((B_PER_W,), jnp.int32),
            pltpu.VMEM((B_PER_W,), jnp.int32),
            pltpu.VMEM((2, CHUNK, DIM), jnp.float32),
            pltpu.VMEM((2, CHUNK, DIM), jnp.float32),
            pltpu.VMEM((2, CHUNK, DIM), jnp.float32),
            pltpu.VMEM((2, CHUNK), jnp.float32),
            pltpu.SemaphoreType.DMA((2,)),
            pltpu.SemaphoreType.DMA,
            pltpu.SemaphoreType.DMA,
        ],
    )
    return run(ent_embs, rel_embs, h_idx, r_idx, t_idx)


def kernel(data, ent_embs, rel_embs):
    h_idx = data[:, 0].astype(jnp.int32)
    r_idx = data[:, 1].astype(jnp.int32)
    t_idx = data[:, 2].astype(jnp.int32)
    return _dist_mult(ent_embs, rel_embs, h_idx, r_idx, t_idx)
